# Initial kernel scaffold; baseline (speedup 1.0000x reference)
import functools

import jax
import jax.numpy as jnp
from jax import lax
from jax.experimental import pallas as pl
from jax.experimental.pallas import tpu as pltpu
from jax.experimental.pallas import tpu_sc as plsc

N = 10000
E = 320000
D = 128
NC = 2
NS = 16
NW = NC * NS
EP = E // NW
K = 80
NCH = EP // K
RPS = N // NS
RB = 125

_mesh = plsc.VectorSubcoreMesh(core_axis_name="c", subcore_axis_name="s")


def _cc_linear_block(xb, W, s, wa=None):
    h = lax.dot_general(xb, W, (((1,), (1,)), ((), ())),
                        preferred_element_type=jnp.float32)
    h0 = h[:, 0:1]
    time = jax.nn.sigmoid(h0) * jnp.exp(s) + 1.1
    sqn = jnp.clip(jnp.sum(h * h, axis=1, keepdims=True) - h0 * h0, 1e-8, None)
    ss = jnp.sqrt((time * time - 1.0) / sqn)
    col = lax.broadcasted_iota(jnp.int32, h.shape, 1)
    out = jnp.where(col == 0, time, h * ss)
    if wa is None:
        return out
    return out, jnp.sum(out * wa, axis=1, keepdims=True)


def _tc1_body(x_ref, wq_ref, wk_ref, wv_ref, wa_ref, sq_ref, sk_ref, sv_ref,
              v_ref, aq_ref, ak_ref):
    xb = x_ref[...]
    wa = wa_ref[...]
    _, aq = _cc_linear_block(xb, wq_ref[...], sq_ref[0, 0], wa[:, :D])
    _, ak = _cc_linear_block(xb, wk_ref[...], sk_ref[0, 0], wa[:, D:])
    v_ref[...] = _cc_linear_block(xb, wv_ref[...], sv_ref[0, 0])
    aq_ref[...] = aq
    ak_ref[...] = ak


def _tc2_body(p_ref, wp_ref, sp_ref, out_ref):
    o = p_ref[0] + p_ref[1]
    o0 = o[:, 0:1]
    inner = jnp.sum(o * o, axis=1, keepdims=True) - 2.0 * o0 * o0
    dn = jnp.sqrt(jnp.clip(jnp.abs(inner), 1e-8, None))
    out_ref[...] = _cc_linear_block(o / dn, wp_ref[...], sp_ref[0, 0])


def _sc_pass_a(src_hbm, dst_hbm, aq_hbm, ak_hbm, ex_hbm, denp_hbm,
               aq_v, ak_v, den_v, src_v, dst_v, exb_v, den_sh):
    cid = lax.axis_index("c")
    sid = lax.axis_index("s")
    wid = sid * NC + cid
    base = wid * EP
    pltpu.sync_copy(aq_hbm, aq_v)
    pltpu.sync_copy(ak_hbm, ak_v)

    def zero_den(i, carry):
        den_v[pl.ds(i * 16, 16)] = jnp.zeros((16,), jnp.float32)
        return carry
    lax.fori_loop(0, N // 16, zero_den, 0)

    @pl.when(sid == 0)
    def _():
        pltpu.sync_copy(den_v, den_sh)
    plsc.subcore_barrier()

    def chunk(i, carry):
        off = pl.multiple_of(base + i * K, 8)
        pltpu.sync_copy(src_hbm.at[pl.ds(off, K)], src_v)
        pltpu.sync_copy(dst_hbm.at[pl.ds(off, K)], dst_v)
        for j in range(K // 16):
            sl = pl.ds(j * 16, 16)
            sv = src_v[sl]
            dv = dst_v[sl]
            a1 = plsc.load_gather(aq_v, [sv])
            a2 = plsc.load_gather(ak_v, [dv])
            s_ = a1 + a2
            s_ = jnp.where(s_ > 0.0, s_, 0.01 * s_)
            exv = jnp.exp(s_)
            exb_v[sl] = exv
            plsc.addupdate_scatter(den_v, [sv], exv)
        pltpu.sync_copy(exb_v, ex_hbm.at[pl.ds(off, K)])
        return carry
    lax.fori_loop(0, NCH, chunk, 0)

    pltpu.sync_copy(den_v, den_sh, add=True)
    plsc.subcore_barrier()

    @pl.when(sid == 0)
    def _():
        pltpu.sync_copy(den_sh, den_v)
        pltpu.sync_copy(den_v, denp_hbm.at[cid])


def _sc_pass_b(src_hbm, dst_hbm, ex_hbm, denp_hbm, v_hbm, outp_hbm,
               den_v, den2_v, src_v, dst_v, exb_v, alp_v, rows_v, z_v,
               out_sh, sem):
    cid = lax.axis_index("c")
    sid = lax.axis_index("s")
    wid = sid * NC + cid
    base = wid * EP
    pltpu.sync_copy(denp_hbm.at[0], den_v)
    pltpu.sync_copy(denp_hbm.at[1], den2_v)

    def add_den(i, carry):
        sl = pl.ds(i * 16, 16)
        den_v[sl] = den_v[sl] + den2_v[sl]
        return carry
    lax.fori_loop(0, N // 16, add_den, 0)

    def zero_z(i, carry):
        r = i // (D // 16)
        c = i % (D // 16)
        z_v[r, pl.ds(c * 16, 16)] = jnp.zeros((16,), jnp.float32)
        return carry
    lax.fori_loop(0, RB * (D // 16), zero_z, 0)
    for t in range(RPS // RB):
        pltpu.sync_copy(z_v, out_sh.at[pl.ds(sid * RPS + t * RB, RB)])
    plsc.subcore_barrier()

    def chunk(i, carry):
        off = pl.multiple_of(base + i * K, 8)
        pltpu.sync_copy(src_hbm.at[pl.ds(off, K)], src_v)
        pltpu.sync_copy(dst_hbm.at[pl.ds(off, K)], dst_v)
        pltpu.sync_copy(ex_hbm.at[pl.ds(off, K)], exb_v)
        pltpu.async_copy(v_hbm.at[dst_v], rows_v, sem).wait()
        for j in range(K // 16):
            sl = pl.ds(j * 16, 16)
            sv = src_v[sl]
            dnv = plsc.load_gather(den_v, [sv])
            alp_v[sl] = exb_v[sl] / (dnv + 1e-16)

        def scale(r, carry2):
            av = plsc.load_gather(alp_v, [jnp.full((16,), r, jnp.int32)])
            for c in range(D // 16):
                sl2 = pl.ds(c * 16, 16)
                rows_v[r, sl2] = rows_v[r, sl2] * av
            return carry2
        lax.fori_loop(0, K, scale, 0)
        pltpu.sync_copy(rows_v, out_sh.at[src_v], add=True)
        return carry
    lax.fori_loop(0, NCH, chunk, 0)

    plsc.subcore_barrier()
    for t in range(RPS // RB):
        sl = pl.ds(sid * RPS + t * RB, RB)
        pltpu.sync_copy(out_sh.at[sl], z_v)
        pltpu.sync_copy(z_v, outp_hbm.at[cid, sl])


def kernel(x_q, edge_index, Wq, Wk, Wv, Wproj, Wattn, sq, sk, sv, sproj):
    n, d = x_q.shape
    src = edge_index[0]
    dst = edge_index[1]
    sq2 = jnp.reshape(sq, (1, 1))
    sk2 = jnp.reshape(sk, (1, 1))
    sv2 = jnp.reshape(sv, (1, 1))
    sp2 = jnp.reshape(sproj, (1, 1))

    R = 1000
    grid = (n // R,)
    full = pl.BlockSpec((d, d), lambda i: (0, 0))
    scal = pl.BlockSpec((1, 1), lambda i: (0, 0))
    v, aq2, ak2 = pl.pallas_call(
        _tc1_body,
        grid=grid,
        in_specs=[
            pl.BlockSpec((R, d), lambda i: (i, 0)),
            full, full, full,
            pl.BlockSpec((1, 2 * d), lambda i: (0, 0)),
            scal, scal, scal,
        ],
        out_specs=[
            pl.BlockSpec((R, d), lambda i: (i, 0)),
            pl.BlockSpec((R, 1), lambda i: (i, 0)),
            pl.BlockSpec((R, 1), lambda i: (i, 0)),
        ],
        out_shape=[
            jax.ShapeDtypeStruct((n, d), jnp.float32),
            jax.ShapeDtypeStruct((n, 1), jnp.float32),
            jax.ShapeDtypeStruct((n, 1), jnp.float32),
        ],
    )(x_q, Wq, Wk, Wv, Wattn, sq2, sk2, sv2)
    aq = aq2.reshape(n)
    ak = ak2.reshape(n)

    ex, denp = pl.kernel(
        _sc_pass_a,
        out_type=[
            jax.ShapeDtypeStruct((E,), jnp.float32),
            jax.ShapeDtypeStruct((NC, N), jnp.float32),
        ],
        mesh=_mesh,
        scratch_types=[
            pltpu.VMEM((N,), jnp.float32),
            pltpu.VMEM((N,), jnp.float32),
            pltpu.VMEM((N,), jnp.float32),
            pltpu.VMEM((K,), jnp.int32),
            pltpu.VMEM((K,), jnp.int32),
            pltpu.VMEM((K,), jnp.float32),
            pltpu.VMEM_SHARED((N,), jnp.float32),
        ],
    )(src, dst, aq, ak)

    outp = pl.kernel(
        _sc_pass_b,
        out_type=jax.ShapeDtypeStruct((NC, N, D), jnp.float32),
        mesh=_mesh,
        scratch_types=[
            pltpu.VMEM((N,), jnp.float32),
            pltpu.VMEM((N,), jnp.float32),
            pltpu.VMEM((K,), jnp.int32),
            pltpu.VMEM((K,), jnp.int32),
            pltpu.VMEM((K,), jnp.float32),
            pltpu.VMEM((K,), jnp.float32),
            pltpu.VMEM((K, D), jnp.float32),
            pltpu.VMEM((RB, D), jnp.float32),
            pltpu.VMEM_SHARED((N, D), jnp.float32),
            pltpu.SemaphoreType.DMA,
        ],
    )(src, dst, ex, denp, v)

    out = pl.pallas_call(
        _tc2_body,
        grid=grid,
        in_specs=[
            pl.BlockSpec((NC, R, d), lambda i: (0, i, 0)),
            full, scal,
        ],
        out_specs=pl.BlockSpec((R, d), lambda i: (i, 0)),
        out_shape=jax.ShapeDtypeStruct((n, d), jnp.float32),
    )(outp, Wproj, sp2)
    return out


# speedup vs baseline: 10.7831x; 10.7831x over previous
import functools

import jax
import jax.numpy as jnp
from jax import lax
from jax.experimental import pallas as pl
from jax.experimental.pallas import tpu as pltpu
from jax.experimental.pallas import tpu_sc as plsc

N = 10000
E = 320000
D = 128
NC = 2
NS = 16
NW = NC * NS
EP = E // NW
K = 80
NCH = EP // K
NP = 10240
SW = NP // NS
RPS = NP // NS
RB = 128

_mesh = plsc.VectorSubcoreMesh(core_axis_name="c", subcore_axis_name="s")
_sc_params = pltpu.CompilerParams(needs_layout_passes=False)


def _cc_linear_block(xb, W, s, wa=None):
    h = lax.dot_general(xb, W, (((1,), (1,)), ((), ())),
                        preferred_element_type=jnp.float32)
    h0 = h[:, 0:1]
    time = jax.nn.sigmoid(h0) * jnp.exp(s) + 1.1
    sqn = jnp.clip(jnp.sum(h * h, axis=1, keepdims=True) - h0 * h0, 1e-8, None)
    ss = jnp.sqrt((time * time - 1.0) / sqn)
    col = lax.broadcasted_iota(jnp.int32, h.shape, 1)
    out = jnp.where(col == 0, time, h * ss)
    if wa is None:
        return out
    return out, jnp.sum(out * wa, axis=1, keepdims=True)


def _tc1_body(x_ref, wq_ref, wk_ref, wv_ref, wa_ref, sq_ref, sk_ref, sv_ref,
              v_ref, aq_ref, ak_ref):
    xb = x_ref[...]
    wa = wa_ref[...]
    _, aq = _cc_linear_block(xb, wq_ref[...], sq_ref[0, 0], wa[:, :D])
    _, ak = _cc_linear_block(xb, wk_ref[...], sk_ref[0, 0], wa[:, D:])
    v_ref[...] = _cc_linear_block(xb, wv_ref[...], sv_ref[0, 0])
    aq_ref[...] = aq
    ak_ref[...] = ak


def _tc2_body(p_ref, wp_ref, sp_ref, out_ref):
    o = p_ref[0] + p_ref[1]
    o0 = o[:, 0:1]
    inner = jnp.sum(o * o, axis=1, keepdims=True) - 2.0 * o0 * o0
    dn = jnp.sqrt(jnp.clip(jnp.abs(inner), 1e-8, None))
    out_ref[...] = _cc_linear_block(o / dn, wp_ref[...], sp_ref[0, 0])


def _sc_pass_a(src_hbm, dst_hbm, aq_hbm, ak_hbm, ex_hbm, denp_hbm,
               aq_v, ak_v, den_v, src_v, dst_v, exb_v, acc_v, tmp_v, den_sh):
    cid = lax.axis_index("c")
    sid = lax.axis_index("s")
    wid = sid * NC + cid
    base = wid * EP
    pltpu.sync_copy(aq_hbm, aq_v)
    pltpu.sync_copy(ak_hbm, ak_v)

    def zero_den(i, carry):
        den_v[pl.ds(i * 16, 16)] = jnp.zeros((16,), jnp.float32)
        return carry
    lax.fori_loop(0, NP // 16, zero_den, 0)

    def chunk(i, carry):
        off = pl.multiple_of(base + i * K, 8)
        pltpu.sync_copy(src_hbm.at[pl.ds(off, K)], src_v)
        pltpu.sync_copy(dst_hbm.at[pl.ds(off, K)], dst_v)
        for j in range(K // 16):
            sl = pl.ds(j * 16, 16)
            sv = src_v[sl]
            dv = dst_v[sl]
            a1 = plsc.load_gather(aq_v, [sv])
            a2 = plsc.load_gather(ak_v, [dv])
            s_ = a1 + a2
            s_ = jnp.where(s_ > 0.0, s_, 0.01 * s_)
            exv = jnp.exp(s_)
            exb_v[sl] = exv
            plsc.addupdate_scatter(den_v, [sv], exv)
        pltpu.sync_copy(exb_v, ex_hbm.at[pl.ds(off, K)])
        return carry
    lax.fori_loop(0, NCH, chunk, 0)

    pltpu.sync_copy(den_v, den_sh.at[sid])
    plsc.subcore_barrier()
    stripe = pl.ds(pl.multiple_of(sid * SW, 8), SW)
    pltpu.sync_copy(den_sh.at[0, stripe], acc_v)

    def red_row(r, carry):
        pltpu.sync_copy(den_sh.at[r, stripe], tmp_v)

        def add_vec(i, carry2):
            sl = pl.ds(i * 16, 16)
            acc_v[sl] = acc_v[sl] + tmp_v[sl]
            return carry2
        lax.fori_loop(0, SW // 16, add_vec, 0)
        return carry
    lax.fori_loop(1, NS, red_row, 0)
    pltpu.sync_copy(acc_v, denp_hbm.at[cid, stripe])


def _sc_pass_b(src_hbm, dst_hbm, ex_hbm, denp_hbm, v_hbm, outp_hbm,
               den_v, den2_v, src_v, dst_v, exb_v, alp_v, rows_v, z_v,
               out_sh, sem):
    cid = lax.axis_index("c")
    sid = lax.axis_index("s")
    wid = sid * NC + cid
    base = wid * EP
    pltpu.sync_copy(denp_hbm.at[0], den_v)
    pltpu.sync_copy(denp_hbm.at[1], den2_v)

    def add_den(i, carry):
        sl = pl.ds(i * 16, 16)
        den_v[sl] = den_v[sl] + den2_v[sl]
        return carry
    lax.fori_loop(0, NP // 16, add_den, 0)

    def zero_z(i, carry):
        r = i // (D // 16)
        c = i % (D // 16)
        z_v[r, pl.ds(c * 16, 16)] = jnp.zeros((16,), jnp.float32)
        return carry
    lax.fori_loop(0, RB * (D // 16), zero_z, 0)
    for t in range(RPS // RB):
        off0 = pl.multiple_of(sid * RPS + t * RB, 8)
        pltpu.sync_copy(z_v, out_sh.at[pl.ds(off0, RB)])
    plsc.subcore_barrier()

    def chunk(i, carry):
        off = pl.multiple_of(base + i * K, 8)
        pltpu.sync_copy(src_hbm.at[pl.ds(off, K)], src_v)
        pltpu.sync_copy(dst_hbm.at[pl.ds(off, K)], dst_v)
        pltpu.sync_copy(ex_hbm.at[pl.ds(off, K)], exb_v)
        pltpu.async_copy(v_hbm.at[dst_v], rows_v, sem).wait()
        for j in range(K // 16):
            sl = pl.ds(j * 16, 16)
            sv = src_v[sl]
            dnv = plsc.load_gather(den_v, [sv])
            alp_v[sl] = exb_v[sl] / (dnv + 1e-16)

        def scale(r, carry2):
            av = plsc.load_gather(alp_v, [jnp.full((16,), r, jnp.int32)])
            for c in range(D // 16):
                sl2 = pl.ds(c * 16, 16)
                rows_v[r, sl2] = rows_v[r, sl2] * av
            return carry2
        lax.fori_loop(0, K, scale, 0)
        pltpu.sync_copy(rows_v, out_sh.at[src_v], add=True)
        return carry
    lax.fori_loop(0, NCH, chunk, 0)

    plsc.subcore_barrier()
    for t in range(RPS // RB):
        sl = pl.ds(pl.multiple_of(sid * RPS + t * RB, 8), RB)
        pltpu.sync_copy(out_sh.at[sl], z_v)
        pltpu.sync_copy(z_v, outp_hbm.at[cid, sl])


def kernel(x_q, edge_index, Wq, Wk, Wv, Wproj, Wattn, sq, sk, sv, sproj):
    n, d = x_q.shape
    src = edge_index[0]
    dst = edge_index[1]
    sq2 = jnp.reshape(sq, (1, 1))
    sk2 = jnp.reshape(sk, (1, 1))
    sv2 = jnp.reshape(sv, (1, 1))
    sp2 = jnp.reshape(sproj, (1, 1))

    R = 1000
    grid = (n // R,)
    full = pl.BlockSpec((d, d), lambda i: (0, 0))
    scal = pl.BlockSpec((1, 1), lambda i: (0, 0))
    v, aq2, ak2 = pl.pallas_call(
        _tc1_body,
        grid=grid,
        in_specs=[
            pl.BlockSpec((R, d), lambda i: (i, 0)),
            full, full, full,
            pl.BlockSpec((1, 2 * d), lambda i: (0, 0)),
            scal, scal, scal,
        ],
        out_specs=[
            pl.BlockSpec((R, d), lambda i: (i, 0)),
            pl.BlockSpec((R, 1), lambda i: (i, 0)),
            pl.BlockSpec((R, 1), lambda i: (i, 0)),
        ],
        out_shape=[
            jax.ShapeDtypeStruct((n, d), jnp.float32),
            jax.ShapeDtypeStruct((n, 1), jnp.float32),
            jax.ShapeDtypeStruct((n, 1), jnp.float32),
        ],
    )(x_q, Wq, Wk, Wv, Wattn, sq2, sk2, sv2)
    pad = jnp.zeros((NP - n,), jnp.float32)
    aq = jnp.concatenate([aq2.reshape(n), pad])
    ak = jnp.concatenate([ak2.reshape(n), pad])

    ex, denp = pl.kernel(
        _sc_pass_a,
        out_type=[
            jax.ShapeDtypeStruct((E,), jnp.float32),
            jax.ShapeDtypeStruct((NC, NP), jnp.float32),
        ],
        mesh=_mesh,
        scratch_types=[
            pltpu.VMEM((NP,), jnp.float32),
            pltpu.VMEM((NP,), jnp.float32),
            pltpu.VMEM((NP,), jnp.float32),
            pltpu.VMEM((K,), jnp.int32),
            pltpu.VMEM((K,), jnp.int32),
            pltpu.VMEM((K,), jnp.float32),
            pltpu.VMEM((SW,), jnp.float32),
            pltpu.VMEM((SW,), jnp.float32),
            pltpu.VMEM_SHARED((NS, NP), jnp.float32),
        ],
        compiler_params=_sc_params,
    )(src, dst, aq, ak)

    outp = pl.kernel(
        _sc_pass_b,
        out_type=jax.ShapeDtypeStruct((NC, NP, D), jnp.float32),
        mesh=_mesh,
        scratch_types=[
            pltpu.VMEM((NP,), jnp.float32),
            pltpu.VMEM((NP,), jnp.float32),
            pltpu.VMEM((K,), jnp.int32),
            pltpu.VMEM((K,), jnp.int32),
            pltpu.VMEM((K,), jnp.float32),
            pltpu.VMEM((128,), jnp.float32),
            pltpu.VMEM((K, D), jnp.float32),
            pltpu.VMEM((RB, D), jnp.float32),
            pltpu.VMEM_SHARED((NP, D), jnp.float32),
            pltpu.SemaphoreType.DMA,
        ],
        compiler_params=_sc_params,
    )(src, dst, ex, denp, v)

    out = pl.pallas_call(
        _tc2_body,
        grid=grid,
        in_specs=[
            pl.BlockSpec((NC, R, d), lambda i: (0, i, 0)),
            full, scal,
        ],
        out_specs=pl.BlockSpec((R, d), lambda i: (i, 0)),
        out_shape=jax.ShapeDtypeStruct((n, d), jnp.float32),
    )(outp, Wproj, sp2)
    return out
